# baseline (device time: 131148 ns/iter reference)
import jax
import jax.numpy as jnp
from jax import lax
from jax.experimental import pallas as pl
from jax.experimental.pallas import tpu as pltpu

T = 2048
D = 1024


def kernel(ids, E):
    V = E.shape[0]

    def body(ids_smem, ids_vmem, e_hbm, out_ref, send_ref, recv_ref,
             gather_sem, send_sem, recv_sem):
        my_x = lax.axis_index("x")
        my_y = lax.axis_index("y")
        my_z = lax.axis_index("z")
        partner = (1 - my_x, my_y, my_z)

        barrier = pltpu.get_barrier_semaphore()
        pl.semaphore_signal(barrier, inc=1, device_id=partner,
                            device_id_type=pl.DeviceIdType.MESH)

        offset = my_x * V

        UNROLL = 8

        def issue(i, carry):
            for j in range(UNROLL):
                t = i * UNROLL + j
                local = ids_smem[t] - offset
                c = jnp.clip(local, 0, V - 1)
                pltpu.make_async_copy(
                    e_hbm.at[pl.ds(c, 1), :],
                    out_ref.at[pl.ds(t, 1), :],
                    gather_sem,
                ).start()
            return carry

        lax.fori_loop(0, T // UNROLL, issue, 0)

        pltpu.make_async_copy(
            e_hbm.at[pl.ds(0, T), :],
            out_ref.at[:, :],
            gather_sem,
        ).wait()

        ids_v = ids_vmem[:, :]
        in_range = (ids_v >= offset) & (ids_v < offset + V)
        masked = jnp.where(in_range, out_ref[:, :], 0.0)
        out_ref[:, :] = masked
        send_ref[:, :] = masked.astype(jnp.bfloat16)

        pl.semaphore_wait(barrier, 1)

        rdma = pltpu.make_async_remote_copy(
            src_ref=send_ref,
            dst_ref=recv_ref,
            send_sem=send_sem,
            recv_sem=recv_sem,
            device_id=partner,
            device_id_type=pl.DeviceIdType.MESH,
        )
        rdma.start()
        rdma.wait()

        out_ref[:, :] = out_ref[:, :] + recv_ref[:, :].astype(jnp.float32)

    ids2 = ids.reshape(T, 1)
    return pl.pallas_call(
        body,
        out_shape=jax.ShapeDtypeStruct((T, D), jnp.float32),
        in_specs=[
            pl.BlockSpec(memory_space=pltpu.SMEM),
            pl.BlockSpec(memory_space=pltpu.VMEM),
            pl.BlockSpec(memory_space=pl.ANY),
        ],
        out_specs=pl.BlockSpec(memory_space=pltpu.VMEM),
        scratch_shapes=[
            pltpu.VMEM((T, D), jnp.bfloat16),
            pltpu.VMEM((T, D), jnp.bfloat16),
            pltpu.SemaphoreType.DMA,
            pltpu.SemaphoreType.DMA,
            pltpu.SemaphoreType.DMA,
        ],
        compiler_params=pltpu.CompilerParams(collective_id=0),
    )(ids, ids2, E)


# device time: 71910 ns/iter; 1.8238x vs baseline; 1.8238x over previous
import jax
import jax.numpy as jnp
from jax import lax
from jax.experimental import pallas as pl
from jax.experimental.pallas import tpu as pltpu

T = 2048
D = 1024
Y = 4
Z = 4
NB = Y * Z
BLK = T // NB
MESH = pl.DeviceIdType.MESH


def kernel(ids, E):
    V = E.shape[0]

    def body(ids_smem, ids_vmem, e_hbm, out_ref,
             own_ref, sendx_ref, recvx_ref, blocks_ref,
             gather_sem, sendx_sem, recvx_sem,
             zr_send, zr_recv, zl_send, zl_recv,
             yr_send, yr_recv, yl_send, yl_recv):
        my_x = lax.axis_index("x")
        my_y = lax.axis_index("y")
        my_z = lax.axis_index("z")
        partner = (1 - my_x, my_y, my_z)
        offset = my_x * V
        b = my_y * Z + my_z
        base = b * BLK

        barrier = pltpu.get_barrier_semaphore()
        pl.semaphore_signal(barrier, inc=1, device_id=partner,
                            device_id_type=MESH)
        for cond, tgt in [
            (my_z > 0, (my_x, my_y, my_z - 1)),
            (my_z < Z - 1, (my_x, my_y, my_z + 1)),
            (my_y > 0, (my_x, my_y - 1, my_z)),
            (my_y < Y - 1, (my_x, my_y + 1, my_z)),
        ]:
            @pl.when(cond)
            def _(tgt=tgt):
                pl.semaphore_signal(barrier, inc=1, device_id=tgt,
                                    device_id_type=MESH)
        n_nbr = (
            1
            + (my_z > 0).astype(jnp.int32) + (my_z < Z - 1).astype(jnp.int32)
            + (my_y > 0).astype(jnp.int32) + (my_y < Y - 1).astype(jnp.int32)
        )

        UNROLL = 4

        def issue(i, carry):
            for j in range(UNROLL):
                t = i * UNROLL + j
                local = ids_smem[base + t] - offset
                c = jnp.clip(local, 0, V - 1)
                pltpu.make_async_copy(
                    e_hbm.at[pl.ds(c, 1), :],
                    own_ref.at[pl.ds(t, 1), :],
                    gather_sem,
                ).start()
            return carry

        lax.fori_loop(0, BLK // UNROLL, issue, 0)
        pltpu.make_async_copy(
            e_hbm.at[pl.ds(0, BLK), :], own_ref.at[:, :], gather_sem
        ).wait()

        ids_b = ids_vmem[pl.ds(base, BLK), :]
        in_range = (ids_b >= offset) & (ids_b < offset + V)
        masked = jnp.where(in_range, own_ref[:, :], 0.0)
        sendx_ref[:, :] = masked.astype(jnp.bfloat16)

        pl.semaphore_wait(barrier, n_nbr)

        xr = pltpu.make_async_remote_copy(
            src_ref=sendx_ref, dst_ref=recvx_ref,
            send_sem=sendx_sem, recv_sem=recvx_sem,
            device_id=partner, device_id_type=MESH,
        )
        xr.start()
        xr.wait()
        summed = masked + recvx_ref[:, :].astype(jnp.float32)
        blocks_ref[pl.ds(b, 1), :, :] = summed.astype(jnp.bfloat16)[None]

        def line_ag(pos, chunk_at, nbr, r_send, r_recv, l_send, l_recv, n):
            for s in range(n - 1):
                @pl.when((pos < n - 1) & (pos - s >= 0))
                def _():
                    pltpu.make_async_remote_copy(
                        src_ref=chunk_at(pos - s), dst_ref=chunk_at(pos - s),
                        send_sem=r_send.at[s], recv_sem=r_recv.at[s],
                        device_id=nbr(1), device_id_type=MESH,
                    ).start()

                @pl.when((pos > 0) & (pos + s <= n - 1))
                def _():
                    pltpu.make_async_remote_copy(
                        src_ref=chunk_at(pos + s), dst_ref=chunk_at(pos + s),
                        send_sem=l_send.at[s], recv_sem=l_recv.at[s],
                        device_id=nbr(-1), device_id_type=MESH,
                    ).start()

                @pl.when((pos > 0) & (pos - 1 - s >= 0))
                def _():
                    pltpu.make_async_remote_copy(
                        src_ref=chunk_at(pos - 1 - s),
                        dst_ref=chunk_at(pos - 1 - s),
                        send_sem=r_send.at[s], recv_sem=r_recv.at[s],
                        device_id=nbr(-1), device_id_type=MESH,
                    ).wait_recv()

                @pl.when((pos < n - 1) & (pos + 1 + s <= n - 1))
                def _():
                    pltpu.make_async_remote_copy(
                        src_ref=chunk_at(pos + 1 + s),
                        dst_ref=chunk_at(pos + 1 + s),
                        send_sem=l_send.at[s], recv_sem=l_recv.at[s],
                        device_id=nbr(1), device_id_type=MESH,
                    ).wait_recv()

            for s in range(n - 1):
                @pl.when((pos < n - 1) & (pos - s >= 0))
                def _():
                    pltpu.make_async_remote_copy(
                        src_ref=chunk_at(pos - s), dst_ref=chunk_at(pos - s),
                        send_sem=r_send.at[s], recv_sem=r_recv.at[s],
                        device_id=nbr(1), device_id_type=MESH,
                    ).wait_send()

                @pl.when((pos > 0) & (pos + s <= n - 1))
                def _():
                    pltpu.make_async_remote_copy(
                        src_ref=chunk_at(pos + s), dst_ref=chunk_at(pos + s),
                        send_sem=l_send.at[s], recv_sem=l_recv.at[s],
                        device_id=nbr(-1), device_id_type=MESH,
                    ).wait_send()

        line_ag(
            my_z,
            lambda p: blocks_ref.at[my_y * Z + p],
            lambda d: (my_x, my_y, my_z + d),
            zr_send, zr_recv, zl_send, zl_recv, Z,
        )

        line_ag(
            my_y,
            lambda p: blocks_ref.at[pl.ds(p * Z, Z)],
            lambda d: (my_x, my_y + d, my_z),
            yr_send, yr_recv, yl_send, yl_recv, Y,
        )

        for i in range(NB):
            out_ref[pl.ds(i * BLK, BLK), :] = (
                blocks_ref[i].astype(jnp.float32)
            )

    ids2 = ids.reshape(T, 1)
    return pl.pallas_call(
        body,
        out_shape=jax.ShapeDtypeStruct((T, D), jnp.float32),
        in_specs=[
            pl.BlockSpec(memory_space=pltpu.SMEM),
            pl.BlockSpec(memory_space=pltpu.VMEM),
            pl.BlockSpec(memory_space=pl.ANY),
        ],
        out_specs=pl.BlockSpec(memory_space=pltpu.VMEM),
        scratch_shapes=[
            pltpu.VMEM((BLK, D), jnp.float32),
            pltpu.VMEM((BLK, D), jnp.bfloat16),
            pltpu.VMEM((BLK, D), jnp.bfloat16),
            pltpu.VMEM((NB, BLK, D), jnp.bfloat16),
            pltpu.SemaphoreType.DMA,
            pltpu.SemaphoreType.DMA,
            pltpu.SemaphoreType.DMA,
            pltpu.SemaphoreType.DMA((Z - 1,)),
            pltpu.SemaphoreType.DMA((Z - 1,)),
            pltpu.SemaphoreType.DMA((Z - 1,)),
            pltpu.SemaphoreType.DMA((Z - 1,)),
            pltpu.SemaphoreType.DMA((Y - 1,)),
            pltpu.SemaphoreType.DMA((Y - 1,)),
            pltpu.SemaphoreType.DMA((Y - 1,)),
            pltpu.SemaphoreType.DMA((Y - 1,)),
        ],
        compiler_params=pltpu.CompilerParams(collective_id=0),
    )(ids, ids2, E)


# device time: 51056 ns/iter; 2.5687x vs baseline; 1.4085x over previous
import jax
import jax.numpy as jnp
from jax import lax
from jax.experimental import pallas as pl
from jax.experimental.pallas import tpu as pltpu

T = 2048
D = 1024
Y = 4
Z = 4
NB = Y * Z
BLK = T // NB
HALF = BLK // 2
MESH = pl.DeviceIdType.MESH


def kernel(ids, E):
    V = E.shape[0]

    def body(ids_smem, ids_vmem, e_hbm, out_ref,
             own_ref, sendx_ref, recvx_ref, a_ref, b_ref,
             gather_sem, sendx_sem, recvx_sem, p1_sems, p2_sems):
        my_x = lax.axis_index("x")
        my_y = lax.axis_index("y")
        my_z = lax.axis_index("z")
        partner = (1 - my_x, my_y, my_z)
        offset = my_x * V
        b = my_y * Z + my_z
        base = b * BLK

        barrier = pltpu.get_barrier_semaphore()
        pl.semaphore_signal(barrier, inc=1, device_id=partner,
                            device_id_type=MESH)
        for cond, tgt in [
            (my_z > 0, (my_x, my_y, my_z - 1)),
            (my_z < Z - 1, (my_x, my_y, my_z + 1)),
            (my_y > 0, (my_x, my_y - 1, my_z)),
            (my_y < Y - 1, (my_x, my_y + 1, my_z)),
        ]:
            @pl.when(cond)
            def _(tgt=tgt):
                pl.semaphore_signal(barrier, inc=1, device_id=tgt,
                                    device_id_type=MESH)
        n_nbr = (
            1
            + (my_z > 0).astype(jnp.int32) + (my_z < Z - 1).astype(jnp.int32)
            + (my_y > 0).astype(jnp.int32) + (my_y < Y - 1).astype(jnp.int32)
        )

        UNROLL = 4

        def issue(i, carry):
            for j in range(UNROLL):
                t = i * UNROLL + j
                local = ids_smem[base + t] - offset
                c = jnp.clip(local, 0, V - 1)
                pltpu.make_async_copy(
                    e_hbm.at[pl.ds(c, 1), :],
                    own_ref.at[pl.ds(t, 1), :],
                    gather_sem,
                ).start()
            return carry

        lax.fori_loop(0, BLK // UNROLL, issue, 0)
        pltpu.make_async_copy(
            e_hbm.at[pl.ds(0, BLK), :], own_ref.at[:, :], gather_sem
        ).wait()

        ids_b = ids_vmem[pl.ds(base, BLK), :]
        in_range = (ids_b >= offset) & (ids_b < offset + V)
        masked = jnp.where(in_range, own_ref[:, :], 0.0)
        sendx_ref[:, :] = masked.astype(jnp.bfloat16)

        pl.semaphore_wait(barrier, n_nbr)

        xr = pltpu.make_async_remote_copy(
            src_ref=sendx_ref, dst_ref=recvx_ref,
            send_sem=sendx_sem, recv_sem=recvx_sem,
            device_id=partner, device_id_type=MESH,
        )
        xr.start()
        xr.wait()
        summed = (masked + recvx_ref[:, :].astype(jnp.float32)).astype(
            jnp.bfloat16
        )
        a_ref[pl.ds(my_y, 1), pl.ds(my_z, 1), :, :] = summed[:HALF][None, None]
        b_ref[pl.ds(my_z, 1), pl.ds(my_y, 1), :, :] = summed[HALF:][None, None]

        def line_ag_multi(flows, n):
            for s in range(n - 1):
                for pos, chunk_at, nbr, (r_s, r_r, l_s, l_r) in flows:
                    @pl.when((pos < n - 1) & (pos - s >= 0))
                    def _(chunk_at=chunk_at, nbr=nbr, r_s=r_s, r_r=r_r,
                          pos=pos):
                        pltpu.make_async_remote_copy(
                            src_ref=chunk_at(pos - s),
                            dst_ref=chunk_at(pos - s),
                            send_sem=r_s.at[s], recv_sem=r_r.at[s],
                            device_id=nbr(1), device_id_type=MESH,
                        ).start()

                    @pl.when((pos > 0) & (pos + s <= n - 1))
                    def _(chunk_at=chunk_at, nbr=nbr, l_s=l_s, l_r=l_r,
                          pos=pos):
                        pltpu.make_async_remote_copy(
                            src_ref=chunk_at(pos + s),
                            dst_ref=chunk_at(pos + s),
                            send_sem=l_s.at[s], recv_sem=l_r.at[s],
                            device_id=nbr(-1), device_id_type=MESH,
                        ).start()

                for pos, chunk_at, nbr, (r_s, r_r, l_s, l_r) in flows:
                    @pl.when((pos > 0) & (pos - 1 - s >= 0))
                    def _(chunk_at=chunk_at, nbr=nbr, r_s=r_s, r_r=r_r,
                          pos=pos):
                        pltpu.make_async_remote_copy(
                            src_ref=chunk_at(pos - 1 - s),
                            dst_ref=chunk_at(pos - 1 - s),
                            send_sem=r_s.at[s], recv_sem=r_r.at[s],
                            device_id=nbr(-1), device_id_type=MESH,
                        ).wait_recv()

                    @pl.when((pos < n - 1) & (pos + 1 + s <= n - 1))
                    def _(chunk_at=chunk_at, nbr=nbr, l_s=l_s, l_r=l_r,
                          pos=pos):
                        pltpu.make_async_remote_copy(
                            src_ref=chunk_at(pos + 1 + s),
                            dst_ref=chunk_at(pos + 1 + s),
                            send_sem=l_s.at[s], recv_sem=l_r.at[s],
                            device_id=nbr(1), device_id_type=MESH,
                        ).wait_recv()

            for s in range(n - 1):
                for pos, chunk_at, nbr, (r_s, r_r, l_s, l_r) in flows:
                    @pl.when((pos < n - 1) & (pos - s >= 0))
                    def _(chunk_at=chunk_at, nbr=nbr, r_s=r_s, r_r=r_r,
                          pos=pos):
                        pltpu.make_async_remote_copy(
                            src_ref=chunk_at(pos - s),
                            dst_ref=chunk_at(pos - s),
                            send_sem=r_s.at[s], recv_sem=r_r.at[s],
                            device_id=nbr(1), device_id_type=MESH,
                        ).wait_send()

                    @pl.when((pos > 0) & (pos + s <= n - 1))
                    def _(chunk_at=chunk_at, nbr=nbr, l_s=l_s, l_r=l_r,
                          pos=pos):
                        pltpu.make_async_remote_copy(
                            src_ref=chunk_at(pos + s),
                            dst_ref=chunk_at(pos + s),
                            send_sem=l_s.at[s], recv_sem=l_r.at[s],
                            device_id=nbr(-1), device_id_type=MESH,
                        ).wait_send()

        z_nbr = lambda d: (my_x, my_y, my_z + d)
        y_nbr = lambda d: (my_x, my_y + d, my_z)
        sems1 = [tuple(p1_sems.at[i] for i in range(4 * f, 4 * f + 4))
                 for f in range(2)]
        sems2 = [tuple(p2_sems.at[i] for i in range(4 * f, 4 * f + 4))
                 for f in range(2)]

        line_ag_multi([
            (my_z, lambda p: a_ref.at[my_y, p], z_nbr, sems1[0]),
            (my_y, lambda p: b_ref.at[my_z, p], y_nbr, sems1[1]),
        ], Z)

        line_ag_multi([
            (my_y, lambda p: a_ref.at[p], y_nbr, sems2[0]),
            (my_z, lambda p: b_ref.at[p], z_nbr, sems2[1]),
        ], Y)

        for yy in range(Y):
            for zz in range(Z):
                r0 = (yy * Z + zz) * BLK
                out_ref[pl.ds(r0, HALF), :] = (
                    a_ref[yy, zz].astype(jnp.float32)
                )
                out_ref[pl.ds(r0 + HALF, HALF), :] = (
                    b_ref[zz, yy].astype(jnp.float32)
                )

    ids2 = ids.reshape(T, 1)
    return pl.pallas_call(
        body,
        out_shape=jax.ShapeDtypeStruct((T, D), jnp.float32),
        in_specs=[
            pl.BlockSpec(memory_space=pltpu.SMEM),
            pl.BlockSpec(memory_space=pltpu.VMEM),
            pl.BlockSpec(memory_space=pl.ANY),
        ],
        out_specs=pl.BlockSpec(memory_space=pltpu.VMEM),
        scratch_shapes=[
            pltpu.VMEM((BLK, D), jnp.float32),
            pltpu.VMEM((BLK, D), jnp.bfloat16),
            pltpu.VMEM((BLK, D), jnp.bfloat16),
            pltpu.VMEM((Y, Z, HALF, D), jnp.bfloat16),
            pltpu.VMEM((Z, Y, HALF, D), jnp.bfloat16),
            pltpu.SemaphoreType.DMA,
            pltpu.SemaphoreType.DMA,
            pltpu.SemaphoreType.DMA,
            pltpu.SemaphoreType.DMA((8, Z - 1)),
            pltpu.SemaphoreType.DMA((8, Y - 1)),
        ],
        compiler_params=pltpu.CompilerParams(collective_id=0),
    )(ids, ids2, E)


# device time: 48918 ns/iter; 2.6810x vs baseline; 1.0437x over previous
import jax
import jax.numpy as jnp
from jax import lax
from jax.experimental import pallas as pl
from jax.experimental.pallas import tpu as pltpu

T = 2048
D = 1024
Y = 4
Z = 4
NB = Y * Z
BLK = T // NB
HALF = BLK // 2
MESH = pl.DeviceIdType.MESH


def kernel(ids, E):
    V = E.shape[0]

    def body(ids_smem, ids_vmem, e_hbm, out_ref,
             own_ref, sendx_ref, recvx_ref, a_ref, b_ref,
             gather_sem, sendx_sem, recvx_sem, p1_sems, p2_sems):
        my_x = lax.axis_index("x")
        my_y = lax.axis_index("y")
        my_z = lax.axis_index("z")
        partner = (1 - my_x, my_y, my_z)
        offset = my_x * V
        b = my_y * Z + my_z
        base = b * BLK

        barrier = pltpu.get_barrier_semaphore()
        pl.semaphore_signal(barrier, inc=1, device_id=partner,
                            device_id_type=MESH)
        n_nbr = jnp.int32(1)
        for k in range(1, max(Y, Z)):
            for cond, tgt in [
                (my_z - k >= 0, (my_x, my_y, my_z - k)),
                (my_z + k <= Z - 1, (my_x, my_y, my_z + k)),
                (my_y - k >= 0, (my_x, my_y - k, my_z)),
                (my_y + k <= Y - 1, (my_x, my_y + k, my_z)),
            ]:
                @pl.when(cond)
                def _(tgt=tgt):
                    pl.semaphore_signal(barrier, inc=1, device_id=tgt,
                                        device_id_type=MESH)
                n_nbr = n_nbr + cond.astype(jnp.int32)

        UNROLL = 4

        def issue(i, carry):
            for j in range(UNROLL):
                t = i * UNROLL + j
                local = ids_smem[base + t] - offset
                c = jnp.clip(local, 0, V - 1)
                pltpu.make_async_copy(
                    e_hbm.at[pl.ds(c, 1), :],
                    own_ref.at[pl.ds(t, 1), :],
                    gather_sem,
                ).start()
            return carry

        lax.fori_loop(0, BLK // UNROLL, issue, 0)
        pltpu.make_async_copy(
            e_hbm.at[pl.ds(0, BLK), :], own_ref.at[:, :], gather_sem
        ).wait()

        ids_b = ids_vmem[pl.ds(base, BLK), :]
        in_range = (ids_b >= offset) & (ids_b < offset + V)
        masked = jnp.where(in_range, own_ref[:, :], 0.0)
        sendx_ref[:, :] = masked.astype(jnp.bfloat16)

        pl.semaphore_wait(barrier, n_nbr)

        xr = pltpu.make_async_remote_copy(
            src_ref=sendx_ref, dst_ref=recvx_ref,
            send_sem=sendx_sem, recv_sem=recvx_sem,
            device_id=partner, device_id_type=MESH,
        )
        xr.start()
        xr.wait()
        summed = masked + recvx_ref[:, :].astype(jnp.float32)
        summed_bf = summed.astype(jnp.bfloat16)
        a_ref[pl.ds(my_y, 1), pl.ds(my_z, 1), :, :] = (
            summed_bf[:HALF][None, None]
        )
        b_ref[pl.ds(my_z, 1), pl.ds(my_y, 1), :, :] = (
            summed_bf[HALF:][None, None]
        )

        z_nbr = lambda d: (my_x, my_y, my_z + d)
        y_nbr = lambda d: (my_x, my_y + d, my_z)
        a1_chunk = lambda p: a_ref.at[my_y, p]
        b1_chunk = lambda p: b_ref.at[my_z, p]
        a2_chunk = lambda p: a_ref.at[p]
        b2_chunk = lambda p: b_ref.at[p]
        sems1 = [tuple(p1_sems.at[i] for i in range(4 * f, 4 * f + 4))
                 for f in range(2)]
        sems2 = [tuple(p2_sems.at[i] for i in range(4 * f, 4 * f + 4))
                 for f in range(2)]
        flows1 = [(my_z, a1_chunk, z_nbr, sems1[0]),
                  (my_y, b1_chunk, y_nbr, sems1[1])]
        flows2 = [(my_y, a2_chunk, y_nbr, sems2[0]),
                  (my_z, b2_chunk, z_nbr, sems2[1])]


        for pos, chunk_at, nbr, (r_s, r_r, l_s, l_r) in flows1:
            for k in range(1, Z):
                @pl.when(pos + k <= Z - 1)
                def _(k=k, pos=pos, chunk_at=chunk_at, nbr=nbr,
                      r_s=r_s, r_r=r_r):
                    pltpu.make_async_remote_copy(
                        src_ref=chunk_at(pos), dst_ref=chunk_at(pos),
                        send_sem=r_s.at[k - 1], recv_sem=r_r.at[k - 1],
                        device_id=nbr(k), device_id_type=MESH,
                    ).start()

                @pl.when(pos - k >= 0)
                def _(k=k, pos=pos, chunk_at=chunk_at, nbr=nbr,
                      l_s=l_s, l_r=l_r):
                    pltpu.make_async_remote_copy(
                        src_ref=chunk_at(pos), dst_ref=chunk_at(pos),
                        send_sem=l_s.at[k - 1], recv_sem=l_r.at[k - 1],
                        device_id=nbr(-k), device_id_type=MESH,
                    ).start()

        out_ref[pl.ds(base, HALF), :] = summed[:HALF]
        out_ref[pl.ds(base + HALF, HALF), :] = summed[HALF:]

        for pos, chunk_at, nbr, (r_s, r_r, l_s, l_r) in flows1:
            for k in range(1, Z):
                @pl.when(pos - k >= 0)
                def _(k=k, pos=pos, chunk_at=chunk_at, nbr=nbr,
                      r_s=r_s, r_r=r_r):
                    pltpu.make_async_remote_copy(
                        src_ref=chunk_at(pos - k), dst_ref=chunk_at(pos - k),
                        send_sem=r_s.at[k - 1], recv_sem=r_r.at[k - 1],
                        device_id=nbr(-k), device_id_type=MESH,
                    ).wait_recv()

                @pl.when(pos + k <= Z - 1)
                def _(k=k, pos=pos, chunk_at=chunk_at, nbr=nbr,
                      l_s=l_s, l_r=l_r):
                    pltpu.make_async_remote_copy(
                        src_ref=chunk_at(pos + k), dst_ref=chunk_at(pos + k),
                        send_sem=l_s.at[k - 1], recv_sem=l_r.at[k - 1],
                        device_id=nbr(k), device_id_type=MESH,
                    ).wait_recv()

        def conv_a_half(p):
            def go():
                val = a_ref[pl.ds(my_y, 1), pl.ds(p, 1), :, :]
                r0 = my_y * Z * BLK + p * BLK
                out_ref[pl.ds(r0, HALF), :] = (
                    val.reshape(HALF, D).astype(jnp.float32)
                )
            return go

        def conv_b_half(p):
            def go():
                val = b_ref[pl.ds(my_z, 1), pl.ds(p, 1), :, :]
                r0 = p * Z * BLK + my_z * BLK + HALF
                out_ref[pl.ds(r0, HALF), :] = (
                    val.reshape(HALF, D).astype(jnp.float32)
                )
            return go

        def conv_a_strip(p):
            def go():
                for zz in range(Z):
                    val = a_ref[pl.ds(p, 1), pl.ds(zz, 1), :, :]
                    r0 = p * Z * BLK + zz * BLK
                    out_ref[pl.ds(r0, HALF), :] = (
                        val.reshape(HALF, D).astype(jnp.float32)
                    )
            return go

        def conv_b_strip(p):
            def go():
                for yy in range(Y):
                    val = b_ref[pl.ds(p, 1), pl.ds(yy, 1), :, :]
                    r0 = yy * Z * BLK + p * BLK + HALF
                    out_ref[pl.ds(r0, HALF), :] = (
                        val.reshape(HALF, D).astype(jnp.float32)
                    )
            return go

        def guarded(cond, fn):
            def go():
                @pl.when(cond)
                def _():
                    fn()
            return go

        pending = []
        for k in range(1, Z):
            pending.append(guarded(my_z - k >= 0, conv_a_half(my_z - k)))
            pending.append(guarded(my_z + k <= Z - 1, conv_a_half(my_z + k)))
            pending.append(guarded(my_y - k >= 0, conv_b_half(my_y - k)))
            pending.append(guarded(my_y + k <= Y - 1, conv_b_half(my_y + k)))

        for s in range(Y - 1):
            for pos, chunk_at, nbr, (r_s, r_r, l_s, l_r) in flows2:
                @pl.when((pos < Y - 1) & (pos - s >= 0))
                def _(s=s, pos=pos, chunk_at=chunk_at, nbr=nbr,
                      r_s=r_s, r_r=r_r):
                    pltpu.make_async_remote_copy(
                        src_ref=chunk_at(pos - s), dst_ref=chunk_at(pos - s),
                        send_sem=r_s.at[s], recv_sem=r_r.at[s],
                        device_id=nbr(1), device_id_type=MESH,
                    ).start()

                @pl.when((pos > 0) & (pos + s <= Y - 1))
                def _(s=s, pos=pos, chunk_at=chunk_at, nbr=nbr,
                      l_s=l_s, l_r=l_r):
                    pltpu.make_async_remote_copy(
                        src_ref=chunk_at(pos + s), dst_ref=chunk_at(pos + s),
                        send_sem=l_s.at[s], recv_sem=l_r.at[s],
                        device_id=nbr(-1), device_id_type=MESH,
                    ).start()

            for fn in pending:
                fn()
            pending = []

            for pos, chunk_at, nbr, (r_s, r_r, l_s, l_r) in flows2:
                @pl.when((pos > 0) & (pos - 1 - s >= 0))
                def _(s=s, pos=pos, chunk_at=chunk_at, nbr=nbr,
                      r_s=r_s, r_r=r_r):
                    pltpu.make_async_remote_copy(
                        src_ref=chunk_at(pos - 1 - s),
                        dst_ref=chunk_at(pos - 1 - s),
                        send_sem=r_s.at[s], recv_sem=r_r.at[s],
                        device_id=nbr(-1), device_id_type=MESH,
                    ).wait_recv()

                @pl.when((pos < Y - 1) & (pos + 1 + s <= Y - 1))
                def _(s=s, pos=pos, chunk_at=chunk_at, nbr=nbr,
                      l_s=l_s, l_r=l_r):
                    pltpu.make_async_remote_copy(
                        src_ref=chunk_at(pos + 1 + s),
                        dst_ref=chunk_at(pos + 1 + s),
                        send_sem=l_s.at[s], recv_sem=l_r.at[s],
                        device_id=nbr(1), device_id_type=MESH,
                    ).wait_recv()

            pending.append(guarded((my_y > 0) & (my_y - 1 - s >= 0),
                                   conv_a_strip(my_y - 1 - s)))
            pending.append(guarded((my_y < Y - 1) & (my_y + 1 + s <= Y - 1),
                                   conv_a_strip(my_y + 1 + s)))
            pending.append(guarded((my_z > 0) & (my_z - 1 - s >= 0),
                                   conv_b_strip(my_z - 1 - s)))
            pending.append(guarded((my_z < Z - 1) & (my_z + 1 + s <= Z - 1),
                                   conv_b_strip(my_z + 1 + s)))

        for fn in pending:
            fn()

        for pos, chunk_at, nbr, (r_s, r_r, l_s, l_r) in flows1:
            for k in range(1, Z):
                @pl.when(pos + k <= Z - 1)
                def _(k=k, pos=pos, chunk_at=chunk_at, nbr=nbr,
                      r_s=r_s, r_r=r_r):
                    pltpu.make_async_remote_copy(
                        src_ref=chunk_at(pos), dst_ref=chunk_at(pos),
                        send_sem=r_s.at[k - 1], recv_sem=r_r.at[k - 1],
                        device_id=nbr(k), device_id_type=MESH,
                    ).wait_send()

                @pl.when(pos - k >= 0)
                def _(k=k, pos=pos, chunk_at=chunk_at, nbr=nbr,
                      l_s=l_s, l_r=l_r):
                    pltpu.make_async_remote_copy(
                        src_ref=chunk_at(pos), dst_ref=chunk_at(pos),
                        send_sem=l_s.at[k - 1], recv_sem=l_r.at[k - 1],
                        device_id=nbr(-k), device_id_type=MESH,
                    ).wait_send()

        for s in range(Y - 1):
            for pos, chunk_at, nbr, (r_s, r_r, l_s, l_r) in flows2:
                @pl.when((pos < Y - 1) & (pos - s >= 0))
                def _(s=s, pos=pos, chunk_at=chunk_at, nbr=nbr,
                      r_s=r_s, r_r=r_r):
                    pltpu.make_async_remote_copy(
                        src_ref=chunk_at(pos - s), dst_ref=chunk_at(pos - s),
                        send_sem=r_s.at[s], recv_sem=r_r.at[s],
                        device_id=nbr(1), device_id_type=MESH,
                    ).wait_send()

                @pl.when((pos > 0) & (pos + s <= Y - 1))
                def _(s=s, pos=pos, chunk_at=chunk_at, nbr=nbr,
                      l_s=l_s, l_r=l_r):
                    pltpu.make_async_remote_copy(
                        src_ref=chunk_at(pos + s), dst_ref=chunk_at(pos + s),
                        send_sem=l_s.at[s], recv_sem=l_r.at[s],
                        device_id=nbr(-1), device_id_type=MESH,
                    ).wait_send()

    ids2 = ids.reshape(T, 1)
    return pl.pallas_call(
        body,
        out_shape=jax.ShapeDtypeStruct((T, D), jnp.float32),
        in_specs=[
            pl.BlockSpec(memory_space=pltpu.SMEM),
            pl.BlockSpec(memory_space=pltpu.VMEM),
            pl.BlockSpec(memory_space=pl.ANY),
        ],
        out_specs=pl.BlockSpec(memory_space=pltpu.VMEM),
        scratch_shapes=[
            pltpu.VMEM((BLK, D), jnp.float32),
            pltpu.VMEM((BLK, D), jnp.bfloat16),
            pltpu.VMEM((BLK, D), jnp.bfloat16),
            pltpu.VMEM((Y, Z, HALF, D), jnp.bfloat16),
            pltpu.VMEM((Z, Y, HALF, D), jnp.bfloat16),
            pltpu.SemaphoreType.DMA,
            pltpu.SemaphoreType.DMA,
            pltpu.SemaphoreType.DMA,
            pltpu.SemaphoreType.DMA((8, Z - 1)),
            pltpu.SemaphoreType.DMA((8, Y - 1)),
        ],
        compiler_params=pltpu.CompilerParams(collective_id=0),
    )(ids, ids2, E)
